# Initial kernel scaffold; baseline (speedup 1.0000x reference)
import jax
import jax.numpy as jnp
from jax.experimental import pallas as pl


def kernel(cached_pre_proj, cached_hidden_states, cached_query_states, cached_key_states, cached_value_states, pre_proj, hidden_states, query_states, key_states, value_states, attn_weights, ref_mask):
    raise NotImplementedError("write your pallas kernel here")


# speedup vs baseline: 1.6266x; 1.6266x over previous
import functools

import jax
import jax.numpy as jnp
from jax import lax
from jax.experimental import pallas as pl
from jax.experimental.pallas import tpu as pltpu
from jax.experimental.pallas import tpu_sc as plsc

B, N, DIM, H, R = 8, 577, 768, 12, 2
NB = N - 1
M = R * N

NC, NS = 2, 16
NW = NC * NS
WPB = NW // B
TPW = NB // WPB
CH = 16
NCHUNK = TPW // CH


def _imp_body(attn_ref, out_ref):
    h = pl.program_id(1)
    colsum = jnp.sum(attn_ref[0, 0], axis=0)

    @pl.when(h == 0)
    def _():
        out_ref[0, 0] = colsum

    @pl.when(h != 0)
    def _():
        out_ref[0, 0] = out_ref[0, 0] + colsum


def _importance_raw(attn):
    return pl.pallas_call(
        _imp_body,
        grid=(B, H),
        in_specs=[pl.BlockSpec((1, 1, N, N), lambda b, h: (b, h, 0, 0))],
        out_specs=pl.BlockSpec((1, 1, N), lambda b, h: (b, 0, 0)),
        out_shape=jax.ShapeDtypeStruct((B, 1, N), jnp.float32),
        compiler_params=pltpu.CompilerParams(
            dimension_semantics=("parallel", "arbitrary")),
    )(attn)


def _dec_body(pp_ref, cpp_ref, imp_ref, bias_ref,
              gidx_ref, gsplat_ref, rmap_ref):
    b = pl.program_id(0)
    pp = pp_ref[0]
    cpp = cpp_ref[0]

    pn = pp / (jnp.sqrt(jnp.sum(pp * pp, axis=1, keepdims=True)) + 1e-6)
    cn = cpp / (jnp.sqrt(jnp.sum(cpp * cpp, axis=1, keepdims=True)) + 1e-6)
    sim = lax.dot_general(
        pn, cn, (((1,), (1,)), ((), ())),
        preferred_element_type=jnp.float32,
        precision=lax.Precision.HIGHEST)

    col = lax.broadcasted_iota(jnp.int32, (1, M), 1)
    bias_row = jnp.where(col < N, bias_ref[0, 0, 0], bias_ref[0, 0, 1])
    sim = sim + bias_row

    smax = jnp.max(sim, axis=1)
    iota_m = lax.broadcasted_iota(jnp.int32, (NB, M), 1)
    amax = jnp.min(jnp.where(sim == smax[:, None], iota_m, M), axis=1)

    imp = imp_ref[0, 0] * (1.0 / H)
    imp = imp / (jnp.max(imp) + 1e-6)
    logit = 10.0 * smax - 5.0 * imp - 2.0
    g = jax.nn.sigmoid(logit)

    gidx_ref[0, 0] = b * M + amax
    gsplat_ref[0] = jnp.broadcast_to(g[:, None], (NB, 16))
    rmap_ref[0, 0] = (logit > 0.0).astype(jnp.int32)


def _decision(pp_body, cpp, impb_raw, bias):
    return pl.pallas_call(
        _dec_body,
        grid=(B,),
        in_specs=[
            pl.BlockSpec((1, NB, DIM), lambda b: (b, 0, 0)),
            pl.BlockSpec((1, M, DIM), lambda b: (b, 0, 0)),
            pl.BlockSpec((1, 1, NB), lambda b: (b, 0, 0)),
            pl.BlockSpec((1, 1, R), lambda b: (b, 0, 0)),
        ],
        out_specs=[
            pl.BlockSpec((1, 1, NB), lambda b: (b, 0, 0)),
            pl.BlockSpec((1, NB, 16), lambda b: (b, 0, 0)),
            pl.BlockSpec((1, 1, NB), lambda b: (b, 0, 0)),
        ],
        out_shape=[
            jax.ShapeDtypeStruct((B, 1, NB), jnp.int32),
            jax.ShapeDtypeStruct((B, NB, 16), jnp.float32),
            jax.ShapeDtypeStruct((B, 1, NB), jnp.int32),
        ],
        compiler_params=pltpu.CompilerParams(
            dimension_semantics=("arbitrary",)),
    )(pp_body, cpp, impb_raw, bias)


def _sc_blend_body(c0, c1, c2, c3, c4, f0, f1, f2, f3, f4, gidx, gsplat,
                   o0, o1, o2, o3, o4, idx_v, g_v, u_v, l_v,
                   cached_ids, cur_ids, sem_u, sem_l, sem_o):
    w = lax.axis_index("s") * NC + lax.axis_index("c")
    b = w // WPB
    q = w % WPB
    tok0 = q * TPW
    row0 = b * N + 1 + tok0
    lane = lax.broadcasted_iota(jnp.int32, (16,), 0)

    pltpu.sync_copy(gidx.at[pl.ds(b * NB + tok0, TPW)], idx_v)
    pltpu.sync_copy(gsplat.at[pl.ds((b * NB + tok0) * 16, TPW * 16)], g_v)

    for t, (cref, fref, oref) in enumerate(
            ((c0, f0, o0), (c1, f1, o1), (c2, f2, o2),
             (c3, f3, o3), (c4, f4, o4))):
        @pl.when(q == 0)
        def _():
            cur_ids[...] = jnp.full((16,), b * N, jnp.int32)
            pltpu.async_copy(fref.at[cur_ids], l_v, sem_l).wait()
            pltpu.async_copy(l_v, oref.at[cur_ids], sem_o).wait()

        def chunk_body(c, _, cref=cref, fref=fref, oref=oref):
            base = row0 + c * CH
            cur_ids[...] = base + lane
            cached_ids[...] = idx_v[pl.ds(c * CH, CH)]
            cp_u = pltpu.async_copy(cref.at[cached_ids], u_v, sem_u)
            cp_l = pltpu.async_copy(fref.at[cur_ids], l_v, sem_l)
            cp_u.wait()
            cp_l.wait()

            def row_body(i, _):
                gs = g_v[pl.ds((c * CH + i) * 16, 16)]

                def vec_body(v, _):
                    u = u_v[i, pl.ds(v * 16, 16)]
                    fl = l_v[i, pl.ds(v * 16, 16)]
                    u_v[i, pl.ds(v * 16, 16)] = fl + gs * (u - fl)
                    return 0

                return lax.fori_loop(0, DIM // 16, vec_body, 0, unroll=4)

            lax.fori_loop(0, CH, row_body, 0)
            pltpu.async_copy(u_v, oref.at[cur_ids], sem_o).wait()
            return 0

        lax.fori_loop(0, NCHUNK, chunk_body, 0)


@functools.cache
def _get_sc_blend():
    mesh = plsc.VectorSubcoreMesh(
        core_axis_name="c", subcore_axis_name="s",
        num_cores=NC, num_subcores=NS)
    return pl.kernel(
        _sc_blend_body,
        out_type=[jax.ShapeDtypeStruct((B * N, DIM), jnp.float32)
                  for _ in range(5)],
        mesh=mesh,
        scratch_types=[
            pltpu.VMEM((TPW,), jnp.int32),
            pltpu.VMEM((TPW * 16,), jnp.float32),
            pltpu.VMEM((CH, DIM), jnp.float32),
            pltpu.VMEM((CH, DIM), jnp.float32),
            pltpu.VMEM((CH,), jnp.int32),
            pltpu.VMEM((CH,), jnp.int32),
            pltpu.SemaphoreType.DMA,
            pltpu.SemaphoreType.DMA,
            pltpu.SemaphoreType.DMA,
        ],
    )


def kernel(cached_pre_proj, cached_hidden_states, cached_query_states,
           cached_key_states, cached_value_states, pre_proj, hidden_states,
           query_states, key_states, value_states, attn_weights, ref_mask):
    imp_raw = _importance_raw(attn_weights)
    impb_raw = imp_raw[:, :, 1:]
    pp_body = pre_proj[:, 1:]
    bias = jnp.where(ref_mask, 0.0, -1e9).astype(jnp.float32).reshape(B, 1, R)

    gidx, gsplat, rmap = _decision(pp_body, cached_pre_proj, impb_raw, bias)

    cached = [t.reshape(B * M, DIM) for t in
              (cached_pre_proj, cached_hidden_states, cached_query_states,
               cached_key_states, cached_value_states)]
    cur = [t.reshape(B * N, DIM) for t in
           (pre_proj, hidden_states, query_states, key_states, value_states)]

    outs = _get_sc_blend()(*cached, *cur, gidx.reshape(B * NB),
                           gsplat.reshape(B * NB * 16))
    outs = [o.reshape(B, N, DIM) for o in outs]

    reuse_map = jnp.concatenate(
        [jnp.zeros((B, 1), dtype=bool), rmap.reshape(B, NB) > 0], axis=1)
    return (reuse_map, outs[0], outs[1], outs[2], outs[3], outs[4])
